# Initial kernel scaffold; baseline (speedup 1.0000x reference)
import jax
import jax.numpy as jnp
from jax.experimental import pallas as pl


def kernel(x, edge_index, batch, W1, b1, bn_g, bn_b, lin0_W, lin0_b, conv_W, conv_b, lin1_W, lin1_b, aW, ab, bW, bb, cW, cb):
    raise NotImplementedError("write your pallas kernel here")


# speedup vs baseline: 6.5057x; 6.5057x over previous
import jax
import jax.numpy as jnp
from jax import lax
from jax.experimental import pallas as pl
from jax.experimental.pallas import tpu as pltpu
from jax.experimental.pallas import tpu_sc as plsc

N = 10000
E = 320000
M = 10000
G = 8
D_FEAT = 128
HID = 256
TGT = 128
ATTN_D = 256

NC = 2
NS = 16
NW = NC * NS
EPW = E // NW
CH = 80
NCHUNK = EPW // CH
RPT = 632
DR = 80

_f32 = jnp.float32


def _attn_pool(z, batch2d, aW, ab, bW, bb, cw_row, cb0):
    R = z.shape[0]
    a = jnp.tanh(jnp.dot(z, aW, preferred_element_type=_f32) + ab[None, :])
    gt = jax.nn.sigmoid(jnp.dot(z, bW, preferred_element_type=_f32) + bb[None, :])
    A = jnp.sum(a * gt * cw_row, axis=1, keepdims=True) + cb0
    onehot = batch2d == lax.broadcasted_iota(jnp.int32, (R, G), 1)
    Am = jnp.where(onehot, A, -1e30)
    smax = jnp.max(Am, axis=0, keepdims=True)
    smax_n = jnp.sum(jnp.where(onehot, smax, 0.0), axis=1, keepdims=True)
    ex = jnp.exp(A - smax_n)
    w = jnp.where(onehot, ex, 0.0)
    den = jnp.sum(w, axis=0)
    den = jnp.where(den > 0.0, den, 1.0)
    num = lax.dot_general(w, z, (((0,), (0,)), ((), ())),
                          preferred_element_type=_f32)
    return num / den[:, None]


def _tc1_body(x_ref, batch_ref, W1_ref, b1_ref, bng_ref, bnb_ref,
              l0W_ref, l0b_ref, convW_ref, l1W_ref,
              aW_ref, ab_ref, bW_ref, bb_ref, cW_ref, cb_ref,
              g_out, pool0_out):
    x = x_ref[...]
    y = jnp.dot(x, W1_ref[...], preferred_element_type=_f32) + b1_ref[...][None, :]
    mu = jnp.mean(y, axis=0, keepdims=True)
    yc = y - mu
    var = jnp.mean(yc * yc, axis=0, keepdims=True)
    h = jnp.maximum(
        yc * lax.rsqrt(var + 1e-5) * bng_ref[...][None, :] + bnb_ref[...][None, :],
        0.0)
    z0 = jnp.dot(h, l0W_ref[...], preferred_element_type=_f32) + l0b_ref[...][None, :]
    Wc = jnp.dot(convW_ref[...], l1W_ref[...], preferred_element_type=_f32)
    g_out[...] = jnp.dot(h, Wc, preferred_element_type=_f32)
    cw_row = cW_ref[...].reshape(1, ATTN_D)
    pool0_out[...] = _attn_pool(z0, batch_ref[...], aW_ref[...], ab_ref[...],
                                bW_ref[...], bb_ref[...], cw_row, cb_ref[0])


def _tcdeg_body(bhp_ref, dnp_ref, binv_out, dn_out):
    bh = jnp.sum(bhp_ref[...], axis=0, keepdims=True)
    binv_out[...] = jnp.where(bh > 0.0, 1.0 / bh, 0.0)
    dn_out[...] = jnp.sum(dnp_ref[...], axis=0, keepdims=True)


def _tccomb_body(pA_ref, binv_ref, u_out):
    u_out[...] = binv_ref[...] * (pA_ref[0] + pA_ref[1])


def _tc2_body(q_ref, dinv_ref, batch_ref, convb_ref, l1W_ref, l1b_ref,
              aW_ref, ab_ref, bW_ref, bb_ref, cW_ref, cb_ref, pool0_ref,
              out_ref):
    s = q_ref[0] + q_ref[1]
    dn = dinv_ref[...]
    dinv = jnp.where(dn > 0.0, 1.0 / dn, 0.0)
    const_row = (jnp.dot(convb_ref[...][None, :], l1W_ref[...],
                         preferred_element_type=_f32) + l1b_ref[...][None, :])
    z1 = dinv * s + const_row
    cw_row = cW_ref[...].reshape(1, ATTN_D)
    pool1 = _attn_pool(z1, batch_ref[...], aW_ref[...], ab_ref[...],
                       bW_ref[...], bb_ref[...], cw_row, cb_ref[0])
    out_ref[...] = pool0_ref[...] + pool1


def _sc_phase_a(g_hbm, ein_hbm, eih_hbm, z_hbm, zf_hbm, pA, bhp_out, dnp_out,
                acc, bhp, dnp, idxn, idxh, rows, sem):
    cid = lax.axis_index("c")
    sid = lax.axis_index("s")
    wid = cid * NS + sid

    r0 = pl.multiple_of(jnp.minimum(sid * RPT, M - RPT), 8)
    pltpu.sync_copy(z_hbm.at[pl.ds(r0, RPT)], acc.at[pl.ds(r0, RPT)])
    pltpu.sync_copy(zf_hbm, bhp)
    pltpu.sync_copy(zf_hbm, dnp)
    plsc.subcore_barrier()

    base = wid * EPW
    ones16 = jnp.ones((16,), _f32)

    @pl.loop(0, NCHUNK)
    def _edges(i):
        off = pl.multiple_of(base + i * CH, 8)
        pltpu.sync_copy(ein_hbm.at[pl.ds(off, CH)], idxn)
        pltpu.sync_copy(eih_hbm.at[pl.ds(off, CH)], idxh)
        pltpu.async_copy(g_hbm.at[idxn], rows, sem).wait()
        pltpu.sync_copy(rows, acc.at[idxh], add=True)
        for k in range(CH // 16):
            vh = idxh[pl.ds(k * 16, 16)]
            plsc.addupdate_scatter(bhp, [vh], ones16)
            vn = idxn[pl.ds(k * 16, 16)]
            plsc.addupdate_scatter(dnp, [vn], ones16)

    plsc.subcore_barrier()

    pltpu.sync_copy(acc.at[pl.ds(r0, RPT)], pA.at[cid, pl.ds(r0, RPT)])
    pltpu.sync_copy(bhp, bhp_out.at[wid])
    pltpu.sync_copy(dnp, dnp_out.at[wid])


def _sc_phase_b(u_hbm, ein_hbm, eih_hbm, z_hbm, qB,
                acc, idxn, idxh, rows, sem):
    cid = lax.axis_index("c")
    sid = lax.axis_index("s")
    wid = cid * NS + sid

    r0 = pl.multiple_of(jnp.minimum(sid * RPT, N - RPT), 8)
    pltpu.sync_copy(z_hbm.at[pl.ds(r0, RPT)], acc.at[pl.ds(r0, RPT)])
    plsc.subcore_barrier()

    base = wid * EPW

    @pl.loop(0, NCHUNK)
    def _edges(i):
        off = pl.multiple_of(base + i * CH, 8)
        pltpu.sync_copy(ein_hbm.at[pl.ds(off, CH)], idxn)
        pltpu.sync_copy(eih_hbm.at[pl.ds(off, CH)], idxh)
        pltpu.async_copy(u_hbm.at[idxh], rows, sem).wait()
        pltpu.sync_copy(rows, acc.at[idxn], add=True)

    plsc.subcore_barrier()
    pltpu.sync_copy(acc.at[pl.ds(r0, RPT)], qB.at[cid, pl.ds(r0, RPT)])


def kernel(x, edge_index, batch, W1, b1, bn_g, bn_b, lin0_W, lin0_b,
           conv_W, conv_b, lin1_W, lin1_b, aW, ab, bW, bb, cW, cb):
    batch2d = batch.reshape(N, 1)
    ein = edge_index[0]
    eih = edge_index[1]
    zeros_hbm = jnp.zeros((M, TGT), _f32)
    zflat_hbm = jnp.zeros((DR * 128,), _f32)

    g_arr, pool0 = pl.pallas_call(
        _tc1_body,
        out_shape=[jax.ShapeDtypeStruct((N, TGT), _f32),
                   jax.ShapeDtypeStruct((G, TGT), _f32)],
    )(x, batch2d, W1, b1, bn_g, bn_b, lin0_W, lin0_b, conv_W, lin1_W,
      aW, ab, bW, bb, cW, cb)

    mesh = plsc.VectorSubcoreMesh(core_axis_name="c", subcore_axis_name="s")

    sca = pl.kernel(
        _sc_phase_a,
        out_type=[jax.ShapeDtypeStruct((NC, M, TGT), _f32),
                  jax.ShapeDtypeStruct((NW, DR * 128), _f32),
                  jax.ShapeDtypeStruct((NW, DR * 128), _f32)],
        mesh=mesh,
        scratch_types=[
            pltpu.VMEM_SHARED((M, TGT), _f32),
            pltpu.VMEM((DR * 128,), _f32),
            pltpu.VMEM((DR * 128,), _f32),
            pltpu.VMEM((CH,), jnp.int32),
            pltpu.VMEM((CH,), jnp.int32),
            pltpu.VMEM((CH, TGT), _f32),
            pltpu.SemaphoreType.DMA,
        ],
        compiler_params=pltpu.CompilerParams(needs_layout_passes=False),
    )
    pA, bhp, dnp = sca(g_arr, ein, eih, zeros_hbm, zflat_hbm)

    binv_flat, dn_flat = pl.pallas_call(
        _tcdeg_body,
        out_shape=[jax.ShapeDtypeStruct((1, DR * 128), _f32),
                   jax.ShapeDtypeStruct((1, DR * 128), _f32)],
    )(bhp, dnp)
    binv_col = binv_flat.reshape(DR * 128, 1)[:M]
    dn_col = dn_flat.reshape(DR * 128, 1)[:N]

    u = pl.pallas_call(
        _tccomb_body,
        out_shape=jax.ShapeDtypeStruct((M, TGT), _f32),
    )(pA, binv_col)

    scb = pl.kernel(
        _sc_phase_b,
        out_type=jax.ShapeDtypeStruct((NC, N, TGT), _f32),
        mesh=mesh,
        scratch_types=[
            pltpu.VMEM_SHARED((N, TGT), _f32),
            pltpu.VMEM((CH,), jnp.int32),
            pltpu.VMEM((CH,), jnp.int32),
            pltpu.VMEM((CH, TGT), _f32),
            pltpu.SemaphoreType.DMA,
        ],
    )
    qB = scb(u, ein, eih, zeros_hbm)

    out = pl.pallas_call(
        _tc2_body,
        out_shape=jax.ShapeDtypeStruct((G, TGT), _f32),
    )(qB, dn_col, batch2d, conv_b, lin1_W, lin1_b,
      aW, ab, bW, bb, cW, cb, pool0)
    return out


# speedup vs baseline: 11.4697x; 1.7630x over previous
import jax
import jax.numpy as jnp
from jax import lax
from jax.experimental import pallas as pl
from jax.experimental.pallas import tpu as pltpu
from jax.experimental.pallas import tpu_sc as plsc

N = 10000
E = 320000
M = 10000
G = 8
D_FEAT = 128
HID = 256
TGT = 128
ATTN_D = 256

NC = 2
NS = 16
NW = NC * NS
EPW = E // NW
CH = 80
NCHUNK = EPW // CH
RPT = 632
DR = 80

_f32 = jnp.float32


def _attn_pool(z, batch2d, aW, ab, bW, bb, cw_row, cb0):
    R = z.shape[0]
    a = jnp.tanh(jnp.dot(z, aW, preferred_element_type=_f32) + ab[None, :])
    gt = jax.nn.sigmoid(jnp.dot(z, bW, preferred_element_type=_f32) + bb[None, :])
    A = jnp.sum(a * gt * cw_row, axis=1, keepdims=True) + cb0
    onehot = batch2d == lax.broadcasted_iota(jnp.int32, (R, G), 1)
    Am = jnp.where(onehot, A, -1e30)
    smax = jnp.max(Am, axis=0, keepdims=True)
    smax_n = jnp.sum(jnp.where(onehot, smax, 0.0), axis=1, keepdims=True)
    ex = jnp.exp(A - smax_n)
    w = jnp.where(onehot, ex, 0.0)
    den = jnp.sum(w, axis=0)
    den = jnp.where(den > 0.0, den, 1.0)
    num = lax.dot_general(w, z, (((0,), (0,)), ((), ())),
                          preferred_element_type=_f32)
    return num / den[:, None]


def _tc1_body(x_ref, batch_ref, W1_ref, b1_ref, bng_ref, bnb_ref,
              l0W_ref, l0b_ref, convW_ref, l1W_ref,
              aW_ref, ab_ref, bW_ref, bb_ref, cW_ref, cb_ref,
              g_out, pool0_out):
    x = x_ref[...]
    y = jnp.dot(x, W1_ref[...], preferred_element_type=_f32) + b1_ref[...][None, :]
    mu = jnp.mean(y, axis=0, keepdims=True)
    yc = y - mu
    var = jnp.mean(yc * yc, axis=0, keepdims=True)
    h = jnp.maximum(
        yc * lax.rsqrt(var + 1e-5) * bng_ref[...][None, :] + bnb_ref[...][None, :],
        0.0)
    z0 = jnp.dot(h, l0W_ref[...], preferred_element_type=_f32) + l0b_ref[...][None, :]
    Wc = jnp.dot(convW_ref[...], l1W_ref[...], preferred_element_type=_f32)
    g_out[...] = jnp.dot(h, Wc, preferred_element_type=_f32)
    cw_row = cW_ref[...].reshape(1, ATTN_D)
    pool0_out[...] = _attn_pool(z0, batch_ref[...], aW_ref[...], ab_ref[...],
                                bW_ref[...], bb_ref[...], cw_row, cb_ref[0])


def _tcdeg_body(bhp_ref, dnp_ref, binv_out, dn_out):
    bh = jnp.sum(bhp_ref[...], axis=0, keepdims=True)
    binv_out[...] = jnp.where(bh > 0.0, 1.0 / bh, 0.0)
    dn_out[...] = jnp.sum(dnp_ref[...], axis=0, keepdims=True)


def _tccomb_body(pA_ref, binv_ref, u_out):
    u_out[...] = binv_ref[...] * (pA_ref[0] + pA_ref[1])


def _tc2_body(q_ref, dinv_ref, batch_ref, convb_ref, l1W_ref, l1b_ref,
              aW_ref, ab_ref, bW_ref, bb_ref, cW_ref, cb_ref, pool0_ref,
              out_ref):
    s = q_ref[0] + q_ref[1]
    dn = dinv_ref[...]
    dinv = jnp.where(dn > 0.0, 1.0 / dn, 0.0)
    const_row = (jnp.dot(convb_ref[...][None, :], l1W_ref[...],
                         preferred_element_type=_f32) + l1b_ref[...][None, :])
    z1 = dinv * s + const_row
    cw_row = cW_ref[...].reshape(1, ATTN_D)
    pool1 = _attn_pool(z1, batch_ref[...], aW_ref[...], ab_ref[...],
                       bW_ref[...], bb_ref[...], cw_row, cb_ref[0])
    out_ref[...] = pool0_ref[...] + pool1


def _edge_pipeline(tbl_hbm, gidx_hbm, sidx_hbm, acc, base,
                   gi0, gi1, si0, si1, rows0, rows1,
                   semG0, semG1, semI0, semI1, degrees=None):
    def off(i):
        return pl.multiple_of(base + i * CH, 8)

    def start_idx(i, gib, sib, semIb):
        pltpu.async_copy(gidx_hbm.at[pl.ds(off(i), CH)], gib, semIb)
        pltpu.async_copy(sidx_hbm.at[pl.ds(off(i), CH)], sib, semIb)

    def wait_idx(i, gib, sib, semIb):
        pltpu.make_async_copy(gidx_hbm.at[pl.ds(off(i), CH)], gib, semIb).wait()
        pltpu.make_async_copy(sidx_hbm.at[pl.ds(off(i), CH)], sib, semIb).wait()

    pltpu.sync_copy(gidx_hbm.at[pl.ds(off(0), CH)], gi0)
    pltpu.sync_copy(sidx_hbm.at[pl.ds(off(0), CH)], si0)
    pltpu.async_copy(tbl_hbm.at[gi0], rows0, semG0)
    start_idx(1, gi1, si1, semI1)

    def half(i, gic, sic, gin, sin, rowsc, rowsn, semGc, semGn, semIc, semIn):
        pltpu.make_async_copy(tbl_hbm.at[gic], rowsc, semGc).wait()

        @pl.when(i + 1 < NCHUNK)
        def _():
            wait_idx(i + 1, gin, sin, semIn)
            pltpu.async_copy(tbl_hbm.at[gin], rowsn, semGn)

        pltpu.sync_copy(rowsc, acc.at[sic], add=True)
        if degrees is not None:
            bhp, dnp, ones16 = degrees
            for k in range(CH // 16):
                vh = sic[pl.ds(k * 16, 16)]
                plsc.addupdate_scatter(bhp, [vh], ones16)
                vn = gic[pl.ds(k * 16, 16)]
                plsc.addupdate_scatter(dnp, [vn], ones16)

        @pl.when(i + 2 < NCHUNK)
        def _():
            start_idx(i + 2, gic, sic, semIc)

    @pl.loop(0, (NCHUNK + 1) // 2)
    def _pairs(j):
        i = 2 * j
        half(i, gi0, si0, gi1, si1, rows0, rows1, semG0, semG1, semI0, semI1)

        @pl.when(i + 1 < NCHUNK)
        def _():
            half(i + 1, gi1, si1, gi0, si0, rows1, rows0,
                 semG1, semG0, semI1, semI0)


def _sc_phase_a(g_hbm, ein_hbm, eih_hbm, z_hbm, zf_hbm, pA, bhp_out, dnp_out,
                acc, bhp, dnp, gi0, gi1, si0, si1, rows0, rows1,
                semG0, semG1, semI0, semI1):
    cid = lax.axis_index("c")
    sid = lax.axis_index("s")
    wid = cid * NS + sid

    r0 = pl.multiple_of(jnp.minimum(sid * RPT, M - RPT), 8)
    pltpu.sync_copy(z_hbm.at[pl.ds(r0, RPT)], acc.at[pl.ds(r0, RPT)])
    pltpu.sync_copy(zf_hbm, bhp)
    pltpu.sync_copy(zf_hbm, dnp)
    plsc.subcore_barrier()

    ones16 = jnp.ones((16,), _f32)
    _edge_pipeline(g_hbm, ein_hbm, eih_hbm, acc, wid * EPW,
                   gi0, gi1, si0, si1, rows0, rows1,
                   semG0, semG1, semI0, semI1, degrees=(bhp, dnp, ones16))

    plsc.subcore_barrier()

    pltpu.sync_copy(acc.at[pl.ds(r0, RPT)], pA.at[cid, pl.ds(r0, RPT)])
    pltpu.sync_copy(bhp, bhp_out.at[wid])
    pltpu.sync_copy(dnp, dnp_out.at[wid])


def _sc_phase_b(u_hbm, ein_hbm, eih_hbm, z_hbm, qB,
                acc, gi0, gi1, si0, si1, rows0, rows1,
                semG0, semG1, semI0, semI1):
    cid = lax.axis_index("c")
    sid = lax.axis_index("s")
    wid = cid * NS + sid

    r0 = pl.multiple_of(jnp.minimum(sid * RPT, N - RPT), 8)
    pltpu.sync_copy(z_hbm.at[pl.ds(r0, RPT)], acc.at[pl.ds(r0, RPT)])
    plsc.subcore_barrier()

    _edge_pipeline(u_hbm, eih_hbm, ein_hbm, acc, wid * EPW,
                   gi0, gi1, si0, si1, rows0, rows1,
                   semG0, semG1, semI0, semI1)

    plsc.subcore_barrier()
    pltpu.sync_copy(acc.at[pl.ds(r0, RPT)], qB.at[cid, pl.ds(r0, RPT)])


def kernel(x, edge_index, batch, W1, b1, bn_g, bn_b, lin0_W, lin0_b,
           conv_W, conv_b, lin1_W, lin1_b, aW, ab, bW, bb, cW, cb):
    batch2d = batch.reshape(N, 1)
    ein = edge_index[0]
    eih = edge_index[1]
    zeros_hbm = jnp.zeros((M, TGT), _f32)
    zflat_hbm = jnp.zeros((DR * 128,), _f32)

    g_arr, pool0 = pl.pallas_call(
        _tc1_body,
        out_shape=[jax.ShapeDtypeStruct((N, TGT), _f32),
                   jax.ShapeDtypeStruct((G, TGT), _f32)],
    )(x, batch2d, W1, b1, bn_g, bn_b, lin0_W, lin0_b, conv_W, lin1_W,
      aW, ab, bW, bb, cW, cb)

    mesh = plsc.VectorSubcoreMesh(core_axis_name="c", subcore_axis_name="s")

    sca = pl.kernel(
        _sc_phase_a,
        out_type=[jax.ShapeDtypeStruct((NC, M, TGT), _f32),
                  jax.ShapeDtypeStruct((NW, DR * 128), _f32),
                  jax.ShapeDtypeStruct((NW, DR * 128), _f32)],
        mesh=mesh,
        scratch_types=[
            pltpu.VMEM_SHARED((M, TGT), _f32),
            pltpu.VMEM((DR * 128,), _f32),
            pltpu.VMEM((DR * 128,), _f32),
            pltpu.VMEM((CH,), jnp.int32),
            pltpu.VMEM((CH,), jnp.int32),
            pltpu.VMEM((CH,), jnp.int32),
            pltpu.VMEM((CH,), jnp.int32),
            pltpu.VMEM((CH, TGT), _f32),
            pltpu.VMEM((CH, TGT), _f32),
            pltpu.SemaphoreType.DMA,
            pltpu.SemaphoreType.DMA,
            pltpu.SemaphoreType.DMA,
            pltpu.SemaphoreType.DMA,
        ],
        compiler_params=pltpu.CompilerParams(needs_layout_passes=False),
    )
    pA, bhp, dnp = sca(g_arr, ein, eih, zeros_hbm, zflat_hbm)

    binv_flat, dn_flat = pl.pallas_call(
        _tcdeg_body,
        out_shape=[jax.ShapeDtypeStruct((1, DR * 128), _f32),
                   jax.ShapeDtypeStruct((1, DR * 128), _f32)],
    )(bhp, dnp)
    binv_col = binv_flat.reshape(DR * 128, 1)[:M]
    dn_col = dn_flat.reshape(DR * 128, 1)[:N]

    u = pl.pallas_call(
        _tccomb_body,
        out_shape=jax.ShapeDtypeStruct((M, TGT), _f32),
    )(pA, binv_col)

    scb = pl.kernel(
        _sc_phase_b,
        out_type=jax.ShapeDtypeStruct((NC, N, TGT), _f32),
        mesh=mesh,
        scratch_types=[
            pltpu.VMEM_SHARED((N, TGT), _f32),
            pltpu.VMEM((CH,), jnp.int32),
            pltpu.VMEM((CH,), jnp.int32),
            pltpu.VMEM((CH,), jnp.int32),
            pltpu.VMEM((CH,), jnp.int32),
            pltpu.VMEM((CH, TGT), _f32),
            pltpu.VMEM((CH, TGT), _f32),
            pltpu.SemaphoreType.DMA,
            pltpu.SemaphoreType.DMA,
            pltpu.SemaphoreType.DMA,
            pltpu.SemaphoreType.DMA,
        ],
    )
    qB = scb(u, ein, eih, zeros_hbm)

    out = pl.pallas_call(
        _tc2_body,
        out_shape=jax.ShapeDtypeStruct((G, TGT), _f32),
    )(qB, dn_col, batch2d, conv_b, lin1_W, lin1_b,
      aW, ab, bW, bb, cW, cb, pool0)
    return out


# speedup vs baseline: 12.3685x; 1.0784x over previous
import jax
import jax.numpy as jnp
from jax import lax
from jax.experimental import pallas as pl
from jax.experimental.pallas import tpu as pltpu
from jax.experimental.pallas import tpu_sc as plsc

N = 10000
E = 320000
M = 10000
G = 8
D_FEAT = 128
HID = 256
TGT = 128
ATTN_D = 256

NC = 2
NS = 16
NW = NC * NS
EPW = E // NW
CH = 80
NCHUNK = EPW // CH
RPT = 632
DR = 80

_f32 = jnp.float32


def _attn_pool(z, batch2d, aW, ab, bW, bb, cw_row, cb0):
    R = z.shape[0]
    a = jnp.tanh(jnp.dot(z, aW, preferred_element_type=_f32) + ab[None, :])
    gt = jax.nn.sigmoid(jnp.dot(z, bW, preferred_element_type=_f32) + bb[None, :])
    A = jnp.sum(a * gt * cw_row, axis=1, keepdims=True) + cb0
    onehot = batch2d == lax.broadcasted_iota(jnp.int32, (R, G), 1)
    Am = jnp.where(onehot, A, -1e30)
    smax = jnp.max(Am, axis=0, keepdims=True)
    smax_n = jnp.sum(jnp.where(onehot, smax, 0.0), axis=1, keepdims=True)
    ex = jnp.exp(A - smax_n)
    w = jnp.where(onehot, ex, 0.0)
    den = jnp.sum(w, axis=0)
    den = jnp.where(den > 0.0, den, 1.0)
    num = lax.dot_general(w, z, (((0,), (0,)), ((), ())),
                          preferred_element_type=_f32)
    return num / den[:, None]


def _tc1_body(x_ref, batch_ref, W1_ref, b1_ref, bng_ref, bnb_ref,
              l0W_ref, l0b_ref, convW_ref, l1W_ref,
              aW_ref, ab_ref, bW_ref, bb_ref, cW_ref, cb_ref,
              g_out, pool0_out):
    x = x_ref[...]
    y = jnp.dot(x, W1_ref[...], preferred_element_type=_f32) + b1_ref[...][None, :]
    mu = jnp.mean(y, axis=0, keepdims=True)
    yc = y - mu
    var = jnp.mean(yc * yc, axis=0, keepdims=True)
    h = jnp.maximum(
        yc * lax.rsqrt(var + 1e-5) * bng_ref[...][None, :] + bnb_ref[...][None, :],
        0.0)
    z0 = jnp.dot(h, l0W_ref[...], preferred_element_type=_f32) + l0b_ref[...][None, :]
    Wc = jnp.dot(convW_ref[...], l1W_ref[...], preferred_element_type=_f32)
    g_out[...] = jnp.dot(h, Wc, preferred_element_type=_f32)
    cw_row = cW_ref[...].reshape(1, ATTN_D)
    pool0_out[...] = _attn_pool(z0, batch_ref[...], aW_ref[...], ab_ref[...],
                                bW_ref[...], bb_ref[...], cw_row, cb_ref[0])


def _tcdeg_body(bhp_ref, binv_out):
    bh = jnp.sum(bhp_ref[...], axis=0, keepdims=True)
    binv_out[...] = jnp.where(bh > 0.0, 1.0 / bh, 0.0)


def _tcsum_body(dnp_ref, dn_out):
    dn_out[...] = jnp.sum(dnp_ref[...], axis=0, keepdims=True)


def _tccomb_body(pA_ref, binv_ref, u_out):
    u_out[...] = binv_ref[...] * (pA_ref[0] + pA_ref[1])


def _tc2_body(q_ref, dinv_ref, batch_ref, convb_ref, l1W_ref, l1b_ref,
              aW_ref, ab_ref, bW_ref, bb_ref, cW_ref, cb_ref, pool0_ref,
              out_ref):
    s = q_ref[0] + q_ref[1]
    dn = dinv_ref[...]
    dinv = jnp.where(dn > 0.0, 1.0 / dn, 0.0)
    const_row = (jnp.dot(convb_ref[...][None, :], l1W_ref[...],
                         preferred_element_type=_f32) + l1b_ref[...][None, :])
    z1 = dinv * s + const_row
    cw_row = cW_ref[...].reshape(1, ATTN_D)
    pool1 = _attn_pool(z1, batch_ref[...], aW_ref[...], ab_ref[...],
                       bW_ref[...], bb_ref[...], cw_row, cb_ref[0])
    out_ref[...] = pool0_ref[...] + pool1


def _edge_pipeline(tbl_hbm, gidx_hbm, sidx_hbm, acc, base,
                   gi, si, rows, semG, semI, deg=None):
    def off(i):
        return pl.multiple_of(base + i * CH, 8)

    def start_idx(i, b):
        pltpu.async_copy(gidx_hbm.at[pl.ds(off(i), CH)], gi[b], semI[b])
        pltpu.async_copy(sidx_hbm.at[pl.ds(off(i), CH)], si[b], semI[b])

    def wait_idx(i, b):
        pltpu.make_async_copy(gidx_hbm.at[pl.ds(off(i), CH)], gi[b], semI[b]).wait()
        pltpu.make_async_copy(sidx_hbm.at[pl.ds(off(i), CH)], si[b], semI[b]).wait()

    def start_gather(b):
        pltpu.async_copy(tbl_hbm.at[gi[b]], rows[b], semG[b])

    def wait_gather(b):
        pltpu.make_async_copy(tbl_hbm.at[gi[b]], rows[b], semG[b]).wait()

    pltpu.sync_copy(gidx_hbm.at[pl.ds(off(0), CH)], gi[0])
    pltpu.sync_copy(sidx_hbm.at[pl.ds(off(0), CH)], si[0])
    start_gather(0)
    start_idx(1, 1)
    wait_idx(1, 1)
    start_gather(1)
    start_idx(2, 2)

    def body(i, c, n, nn):
        wait_gather(c)

        @pl.when(i + 2 < NCHUNK)
        def _():
            wait_idx(i + 2, nn)
            start_gather(nn)

        pltpu.sync_copy(rows[c], acc.at[si[c]], add=True)
        if deg is not None:
            dp, ones16 = deg
            for k in range(CH // 16):
                v = si[c][pl.ds(k * 16, 16)]
                plsc.addupdate_scatter(dp, [v], ones16)

        @pl.when(i + 3 < NCHUNK)
        def _():
            start_idx(i + 3, c)

    @pl.loop(0, (NCHUNK + 2) // 3)
    def _triples(j):
        i = 3 * j
        body(i, 0, 1, 2)

        @pl.when(i + 1 < NCHUNK)
        def _():
            body(i + 1, 1, 2, 0)

        @pl.when(i + 2 < NCHUNK)
        def _():
            body(i + 2, 2, 0, 1)


def _sc_phase_a(g_hbm, ein_hbm, eih_hbm, z_hbm, zf_hbm, pA, bhp_out,
                acc, bhp, gi0, gi1, gi2, si0, si1, si2, r0b, r1b, r2b,
                sG0, sG1, sG2, sI0, sI1, sI2):
    cid = lax.axis_index("c")
    sid = lax.axis_index("s")
    wid = cid * NS + sid

    r0 = pl.multiple_of(jnp.minimum(sid * RPT, M - RPT), 8)
    pltpu.sync_copy(z_hbm.at[pl.ds(r0, RPT)], acc.at[pl.ds(r0, RPT)])
    pltpu.sync_copy(zf_hbm, bhp)
    plsc.subcore_barrier()

    ones16 = jnp.ones((16,), _f32)
    _edge_pipeline(g_hbm, ein_hbm, eih_hbm, acc, wid * EPW,
                   (gi0, gi1, gi2), (si0, si1, si2), (r0b, r1b, r2b),
                   (sG0, sG1, sG2), (sI0, sI1, sI2), deg=(bhp, ones16))

    plsc.subcore_barrier()

    pltpu.sync_copy(acc.at[pl.ds(r0, RPT)], pA.at[cid, pl.ds(r0, RPT)])
    pltpu.sync_copy(bhp, bhp_out.at[wid])


def _sc_phase_b(u_hbm, ein_hbm, eih_hbm, z_hbm, zf_hbm, qB, dnp_out,
                acc, dnp, gi0, gi1, gi2, si0, si1, si2, r0b, r1b, r2b,
                sG0, sG1, sG2, sI0, sI1, sI2):
    cid = lax.axis_index("c")
    sid = lax.axis_index("s")
    wid = cid * NS + sid

    r0 = pl.multiple_of(jnp.minimum(sid * RPT, N - RPT), 8)
    pltpu.sync_copy(z_hbm.at[pl.ds(r0, RPT)], acc.at[pl.ds(r0, RPT)])
    pltpu.sync_copy(zf_hbm, dnp)
    plsc.subcore_barrier()

    ones16 = jnp.ones((16,), _f32)
    _edge_pipeline(u_hbm, eih_hbm, ein_hbm, acc, wid * EPW,
                   (gi0, gi1, gi2), (si0, si1, si2), (r0b, r1b, r2b),
                   (sG0, sG1, sG2), (sI0, sI1, sI2), deg=(dnp, ones16))

    plsc.subcore_barrier()
    pltpu.sync_copy(acc.at[pl.ds(r0, RPT)], qB.at[cid, pl.ds(r0, RPT)])
    pltpu.sync_copy(dnp, dnp_out.at[wid])


def kernel(x, edge_index, batch, W1, b1, bn_g, bn_b, lin0_W, lin0_b,
           conv_W, conv_b, lin1_W, lin1_b, aW, ab, bW, bb, cW, cb):
    batch2d = batch.reshape(N, 1)
    ein = edge_index[0]
    eih = edge_index[1]
    zeros_hbm = jnp.zeros((M, TGT), _f32)
    zflat_hbm = jnp.zeros((DR * 128,), _f32)

    g_arr, pool0 = pl.pallas_call(
        _tc1_body,
        out_shape=[jax.ShapeDtypeStruct((N, TGT), _f32),
                   jax.ShapeDtypeStruct((G, TGT), _f32)],
    )(x, batch2d, W1, b1, bn_g, bn_b, lin0_W, lin0_b, conv_W, lin1_W,
      aW, ab, bW, bb, cW, cb)

    mesh = plsc.VectorSubcoreMesh(core_axis_name="c", subcore_axis_name="s")

    sca = pl.kernel(
        _sc_phase_a,
        out_type=[jax.ShapeDtypeStruct((NC, M, TGT), _f32),
                  jax.ShapeDtypeStruct((NW, DR * 128), _f32)],
        mesh=mesh,
        scratch_types=(
            [pltpu.VMEM_SHARED((M, TGT), _f32),
             pltpu.VMEM((DR * 128,), _f32)]
            + [pltpu.VMEM((CH,), jnp.int32)] * 6
            + [pltpu.VMEM((CH, TGT), _f32)] * 3
            + [pltpu.SemaphoreType.DMA] * 6
        ),
        compiler_params=pltpu.CompilerParams(needs_layout_passes=False),
    )
    pA, bhp = sca(g_arr, ein, eih, zeros_hbm, zflat_hbm)

    binv_flat = pl.pallas_call(
        _tcdeg_body,
        out_shape=jax.ShapeDtypeStruct((1, DR * 128), _f32),
    )(bhp)
    binv_col = binv_flat.reshape(DR * 128, 1)[:M]

    u = pl.pallas_call(
        _tccomb_body,
        out_shape=jax.ShapeDtypeStruct((M, TGT), _f32),
    )(pA, binv_col)

    scb = pl.kernel(
        _sc_phase_b,
        out_type=[jax.ShapeDtypeStruct((NC, N, TGT), _f32),
                  jax.ShapeDtypeStruct((NW, DR * 128), _f32)],
        mesh=mesh,
        scratch_types=(
            [pltpu.VMEM_SHARED((N, TGT), _f32),
             pltpu.VMEM((DR * 128,), _f32)]
            + [pltpu.VMEM((CH,), jnp.int32)] * 6
            + [pltpu.VMEM((CH, TGT), _f32)] * 3
            + [pltpu.SemaphoreType.DMA] * 6
        ),
        compiler_params=pltpu.CompilerParams(needs_layout_passes=False),
    )
    qB, dnp = scb(u, ein, eih, zeros_hbm, zflat_hbm)

    dn_flat = pl.pallas_call(
        _tcsum_body,
        out_shape=jax.ShapeDtypeStruct((1, DR * 128), _f32),
    )(dnp)
    dn_col = dn_flat.reshape(DR * 128, 1)[:N]

    out = pl.pallas_call(
        _tc2_body,
        out_shape=jax.ShapeDtypeStruct((G, TGT), _f32),
    )(qB, dn_col, batch2d, conv_b, lin1_W, lin1_b,
      aW, ab, bW, bb, cW, cb, pool0)
    return out
